# Initial kernel scaffold; baseline (speedup 1.0000x reference)
import functools

import jax
import jax.numpy as jnp
from jax import lax
from jax.experimental import pallas as pl
from jax.experimental.pallas import tpu as pltpu
from jax.experimental.pallas import tpu_sc as plsc

N_NODES = 100000
N_EDGES = 1600000
HEADS = 4
HID = 32
NEG_GAT = 0.2
NEG = 0.01

BN = 2048
GRID = 49
NPAD = BN * GRID

NW = 32
EW = 50176
EP = NW * EW
CH = 1024
CROWS = 8
NCH = EW // CH
SHARE = NPAD // 16
ZROWS = SHARE // 8


def _lrelu(v, s):
    return jnp.where(v >= 0, v, s * v)


def _prep_body(x_ref, wsrc_ref, wdst_ref, st_ref, dt_ref):
    xb = x_ref[...]
    asrc = jnp.dot(xb, wsrc_ref[...], preferred_element_type=jnp.float32)
    adst = jnp.dot(xb, wdst_ref[...], preferred_element_type=jnp.float32)
    zero = jnp.zeros((xb.shape[0], 1), jnp.float32)
    st_ref[...] = jnp.concatenate([asrc, xb, zero], axis=1)
    dt_ref[...] = adst


def _prep(x_pad, w_src, w_dst):
    return pl.pallas_call(
        _prep_body,
        grid=(GRID,),
        in_specs=[
            pl.BlockSpec((BN, 3), lambda i: (i, 0)),
            pl.BlockSpec((3, HEADS), lambda i: (0, 0)),
            pl.BlockSpec((3, HEADS), lambda i: (0, 0)),
        ],
        out_specs=[
            pl.BlockSpec((BN, 8), lambda i: (i, 0)),
            pl.BlockSpec((BN, 4), lambda i: (i, 0)),
        ],
        out_shape=[
            jax.ShapeDtypeStruct((NPAD, 8), jnp.float32),
            jax.ShapeDtypeStruct((NPAD, 4), jnp.float32),
        ],
    )(x_pad, w_src, w_dst)


def _edge_body(sidx_hbm, didx_hbm, st_hbm, dt_hbm, out_hbm,
               sidx_v, didx_v, srow_v, drow_v, upd_v, zbuf_v, acc_sh, sem):
    cid = lax.axis_index("c")
    sid = lax.axis_index("s")
    wid = cid * 16 + sid

    @pl.loop(0, ZROWS)
    def _zero_zbuf(i):
        zbuf_v[i] = jnp.zeros((16,), jnp.float32)

    @pl.loop(0, 8)
    def _zero_acc(j):
        pltpu.sync_copy(zbuf_v, acc_sh.at[pl.ds(sid * SHARE + j * ZROWS, ZROWS)])

    plsc.subcore_barrier()

    wrow = wid * (EW // 128)

    @pl.loop(0, NCH)
    def _chunk(ch):
        r0 = wrow + ch * CROWS
        pltpu.sync_copy(sidx_hbm.at[pl.ds(r0, CROWS)], sidx_v)
        pltpu.sync_copy(didx_hbm.at[pl.ds(r0, CROWS)], didx_v)

        descs = []
        for j in range(CROWS):
            descs.append(pltpu.async_copy(
                st_hbm.at[sidx_v.at[j]], srow_v.at[pl.ds(j * 128, 128)], sem))
            descs.append(pltpu.async_copy(
                dt_hbm.at[didx_v.at[j]], drow_v.at[pl.ds(j * 128, 128)], sem))
        for d in descs:
            d.wait()

        @pl.loop(0, CH // 16)
        def _group(g):
            rows = g * 16 + lax.iota(jnp.int32, 16)

            def col(c):
                return jnp.full((16,), c, jnp.int32)

            a_s = [plsc.load_gather(srow_v, [rows, col(c)]) for c in range(4)]
            x_s = [plsc.load_gather(srow_v, [rows, col(4 + d)]) for d in range(3)]
            a_d = [plsc.load_gather(drow_v, [rows, col(c)]) for c in range(4)]
            es = []
            for h in range(4):
                v = a_s[h] + a_d[h]
                e = jnp.exp(jnp.where(v >= 0, v, NEG_GAT * v))
                es.append(e)
                plsc.store_scatter(upd_v, [rows, col(h)], e)
            for h in range(4):
                for d in range(3):
                    plsc.store_scatter(upd_v, [rows, col(4 + h * 3 + d)],
                                       es[h] * x_s[d])

        for j in range(CROWS):
            pltpu.sync_copy(upd_v.at[pl.ds(j * 128, 128)],
                            acc_sh.at[didx_v.at[j]], add=True)

    plsc.subcore_barrier()

    @pl.loop(0, 8)
    def _copy_out(j):
        off = sid * SHARE + j * ZROWS
        pltpu.sync_copy(acc_sh.at[pl.ds(off, ZROWS)],
                        out_hbm.at[cid].at[pl.ds(off, ZROWS)])


def _edges(sidx2d, didx2d, st, dt):
    mesh = plsc.VectorSubcoreMesh(core_axis_name="c", subcore_axis_name="s")
    fn = pl.kernel(
        _edge_body,
        out_type=jax.ShapeDtypeStruct((2, NPAD, 16), jnp.float32),
        mesh=mesh,
        scratch_types=[
            pltpu.VMEM((CROWS, 128), jnp.int32),
            pltpu.VMEM((CROWS, 128), jnp.int32),
            pltpu.VMEM((CH, 8), jnp.float32),
            pltpu.VMEM((CH, 4), jnp.float32),
            pltpu.VMEM((CH, 16), jnp.float32),
            pltpu.VMEM((ZROWS, 16), jnp.float32),
            pltpu.VMEM_SHARED((NPAD, 16), jnp.float32),
            pltpu.SemaphoreType.DMA,
        ],
    )
    return fn(sidx2d, didx2d, st, dt)


def _final_body(p0_ref, p1_ref, st_ref, dt_ref, bg_ref,
                wgh_refs, w1x_refs, w1h_refs, wmid_refs, w5_refs,
                bmid_refs, b5_refs, id_ref, p_ref):
    s = p0_ref[...] + p1_ref[...]
    asrc = st_ref[:, 0:4]
    xb = st_ref[:, 4:7]
    adst = dt_ref[...]
    vl = asrc + adst
    eal = jnp.exp(_lrelu(vl, NEG_GAT))
    denom = s[:, 0:4] + eal + 1e-16

    g = jnp.zeros((s.shape[0], HID), jnp.float32)
    for h in range(4):
        acc_h = s[:, 4 + 3 * h:7 + 3 * h] + eal[:, h:h + 1] * xb
        num_h = jnp.dot(acc_h, wgh_refs[h][...],
                        preferred_element_type=jnp.float32)
        g = g + num_h / denom[:, h:h + 1]
    g = g * 0.25 + bg_ref[...]
    hfeat = _lrelu(g, NEG)

    for m in range(2):
        u = jnp.dot(xb, w1x_refs[m][...], preferred_element_type=jnp.float32)
        u = u + jnp.dot(hfeat, w1h_refs[m][...],
                        preferred_element_type=jnp.float32)
        u = _lrelu(u + bmid_refs[m][:, 0:HID], NEG)
        for l in range(3):
            u = jnp.dot(u, wmid_refs[m][l][...],
                        preferred_element_type=jnp.float32)
            u = _lrelu(u + bmid_refs[m][:, (l + 1) * HID:(l + 2) * HID], NEG)
        u = jnp.dot(u, w5_refs[m][...], preferred_element_type=jnp.float32)
        u = u + b5_refs[m][...]
        if m == 0:
            id_ref[...] = u
        else:
            p_ref[...] = u


def _final_wrapped(p0_ref, p1_ref, st_ref, dt_ref, bg_ref,
                   wgh0, wgh1, wgh2, wgh3,
                   w1x_0, w1h_0, wm0_0, wm1_0, wm2_0, w5_0, bm_0, b5_0,
                   w1x_1, w1h_1, wm0_1, wm1_1, wm2_1, w5_1, bm_1, b5_1,
                   id_ref, p_ref):
    _final_body(p0_ref, p1_ref, st_ref, dt_ref, bg_ref,
                [wgh0, wgh1, wgh2, wgh3],
                [w1x_0, w1x_1], [w1h_0, w1h_1],
                [[wm0_0, wm1_0, wm2_0], [wm0_1, wm1_1, wm2_1]],
                [w5_0, w5_1], [bm_0, bm_1], [b5_0, b5_1],
                id_ref, p_ref)


def _finalize(p0, p1, st, dt, bg, wghs, mlp1, mlp2):
    def fixed(shape):
        nd = len(shape)
        return pl.BlockSpec(shape, lambda i, _nd=nd: (0,) * _nd)

    in_specs = [
        pl.BlockSpec((BN, 16), lambda i: (i, 0)),
        pl.BlockSpec((BN, 16), lambda i: (i, 0)),
        pl.BlockSpec((BN, 8), lambda i: (i, 0)),
        pl.BlockSpec((BN, 4), lambda i: (i, 0)),
        fixed((1, HID)),
    ]
    args = [p0, p1, st, dt, bg]
    for w in wghs:
        in_specs.append(fixed((3, HID)))
        args.append(w)
    for m in (mlp1, mlp2):
        w1x, w1h, wm0, wm1, wm2, w5, bm, b5 = m
        in_specs += [fixed((3, HID)), fixed((HID, HID)),
                     fixed((HID, HID)), fixed((HID, HID)), fixed((HID, HID)),
                     fixed((HID, w5.shape[1])),
                     fixed((1, 4 * HID)), fixed((1, b5.shape[1]))]
        args += [w1x, w1h, wm0, wm1, wm2, w5, bm, b5]

    return pl.pallas_call(
        _final_wrapped,
        grid=(GRID,),
        in_specs=in_specs,
        out_specs=[
            pl.BlockSpec((BN, 8), lambda i: (i, 0)),
            pl.BlockSpec((BN, 3), lambda i: (i, 0)),
        ],
        out_shape=[
            jax.ShapeDtypeStruct((NPAD, 8), jnp.float32),
            jax.ShapeDtypeStruct((NPAD, 3), jnp.float32),
        ],
    )(*args)


def kernel(x, edge_index, edge_attr, W_gat, att_src, att_dst, b_gat,
           nn1_Ws, nn1_bs, nn2_Ws, nn2_bs):
    del edge_attr

    W3 = W_gat.reshape(3, HEADS, HID)
    w_src = jnp.einsum("dhk,hk->dh", W3, att_src)
    w_dst = jnp.einsum("dhk,hk->dh", W3, att_dst)

    x_pad = jnp.zeros((NPAD, 3), jnp.float32).at[:N_NODES].set(x)
    st, dt = _prep(x_pad, w_src, w_dst)

    pad = jnp.full((EP - N_EDGES,), N_NODES, jnp.int32)
    sidx2d = jnp.concatenate([edge_index[0], pad]).reshape(EP // 128, 128)
    didx2d = jnp.concatenate([edge_index[1], pad]).reshape(EP // 128, 128)

    partials = _edges(sidx2d, didx2d, st, dt)

    def prep_mlp(Ws, bs):
        w1x = Ws[0][0:3]
        w1h = Ws[0][3:3 + HID]
        bm = jnp.concatenate([bs[0], bs[1], bs[2], bs[3]]).reshape(1, 4 * HID)
        b5 = bs[4].reshape(1, -1)
        return (w1x, w1h, Ws[1], Ws[2], Ws[3], Ws[4], bm, b5)

    wghs = [W_gat[:, h * HID:(h + 1) * HID] for h in range(HEADS)]
    id_pad, p_pad = _finalize(
        partials[0], partials[1], st, dt, b_gat.reshape(1, HID), wghs,
        prep_mlp(nn1_Ws, nn1_bs), prep_mlp(nn2_Ws, nn2_bs))

    return (id_pad[:N_NODES], p_pad[:N_NODES])


# speedup vs baseline: 154.4253x; 154.4253x over previous
import dataclasses
import functools

import jax
import jax.numpy as jnp
from jax import lax
from jax.experimental import pallas as pl
from jax.experimental.pallas import tpu as pltpu
from jax.experimental.pallas import tpu_sc as plsc

N_NODES = 100000
N_EDGES = 1600000
HEADS = 4
HID = 32
NEG_GAT = 0.2
NEG = 0.01

BN = 2048
GRID = 49
NPAD = BN * GRID

NW = 32
EP = 1605632
CH = 1024
CROWS = 8
EW = EP // 16
NCH = EW // CH
HALF = NPAD // 2
ACC_ROWS = HALF + 128
SHARE = ACC_ROWS // 16
ZROWS = SHARE // 8
OSHARE = HALF // 16


def _lrelu(v, s):
    return jnp.where(v >= 0, v, s * v)


def _prep_body(x_ref, wsrc_ref, wdst_ref, st_ref, dt_ref):
    xb = x_ref[...]
    asrc = jnp.dot(xb, wsrc_ref[...], preferred_element_type=jnp.float32)
    adst = jnp.dot(xb, wdst_ref[...], preferred_element_type=jnp.float32)
    zero = jnp.zeros((xb.shape[0], 1), jnp.float32)
    st_ref[...] = jnp.concatenate([asrc, xb, zero], axis=1)
    dt_ref[...] = adst


def _prep(x_pad, w_src, w_dst):
    return pl.pallas_call(
        _prep_body,
        grid=(GRID,),
        in_specs=[
            pl.BlockSpec((BN, 3), lambda i: (i, 0)),
            pl.BlockSpec((3, HEADS), lambda i: (0, 0)),
            pl.BlockSpec((3, HEADS), lambda i: (0, 0)),
        ],
        out_specs=[
            pl.BlockSpec((BN, 8), lambda i: (i, 0)),
            pl.BlockSpec((BN, 4), lambda i: (i, 0)),
        ],
        out_shape=[
            jax.ShapeDtypeStruct((NPAD, 8), jnp.float32),
            jax.ShapeDtypeStruct((NPAD, 4), jnp.float32),
        ],
    )(x_pad, w_src, w_dst)


def _edge_body(sidx_hbm, didx_hbm, st_hbm, dt_hbm, out_hbm,
               sidx_v, didx_v, ldx_v, srow_v, drow_v, upd_v, zbuf_v,
               acc_sh, sem):
    cid = lax.axis_index("c")
    sid = lax.axis_index("s")

    @pl.loop(0, ZROWS)
    def _zero_zbuf(i):
        zbuf_v[i] = jnp.zeros((16,), jnp.float32)

    @pl.loop(0, 8)
    def _zero_acc(j):
        pltpu.sync_copy(zbuf_v, acc_sh.at[pl.ds(sid * SHARE + j * ZROWS, ZROWS)])

    plsc.subcore_barrier()

    wrow = sid * (EW // 128)
    base = cid * HALF

    @pl.loop(0, NCH)
    def _chunk(ch):
        r0 = wrow + ch * CROWS
        pltpu.sync_copy(sidx_hbm.at[pl.ds(r0, CROWS)], sidx_v)
        pltpu.sync_copy(didx_hbm.at[pl.ds(r0, CROWS)], didx_v)

        descs = []
        for j in range(CROWS):
            descs.append(pltpu.async_copy(
                st_hbm.at[sidx_v.at[j]], srow_v.at[pl.ds(j * 128, 128)], sem))
            descs.append(pltpu.async_copy(
                dt_hbm.at[didx_v.at[j]], drow_v.at[pl.ds(j * 128, 128)], sem))
        for d in descs:
            d.wait()

        @pl.loop(0, CROWS)
        def _remap(j):
            for k in range(8):
                dd = didx_v.at[j][pl.ds(k * 16, 16)]
                t = dd - base
                ok = (t >= 0) & (t < HALF)
                ldx_v.at[j][pl.ds(k * 16, 16)] = jnp.where(ok, t, HALF)

        @pl.loop(0, CH // 16)
        def _group(g):
            rows = g * 16 + lax.iota(jnp.int32, 16)

            def col(c):
                return jnp.full((16,), c, jnp.int32)

            a_s = [plsc.load_gather(srow_v, [rows, col(c)]) for c in range(4)]
            x_s = [plsc.load_gather(srow_v, [rows, col(4 + d)]) for d in range(3)]
            a_d = [plsc.load_gather(drow_v, [rows, col(c)]) for c in range(4)]
            es = []
            for h in range(4):
                v = a_s[h] + a_d[h]
                e = jnp.exp(jnp.where(v >= 0, v, NEG_GAT * v))
                es.append(e)
                plsc.store_scatter(upd_v, [rows, col(h)], e)
            for h in range(4):
                for d in range(3):
                    plsc.store_scatter(upd_v, [rows, col(4 + h * 3 + d)],
                                       es[h] * x_s[d])

        for j in range(CROWS):
            pltpu.sync_copy(upd_v.at[pl.ds(j * 128, 128)],
                            acc_sh.at[ldx_v.at[j]], add=True)

    plsc.subcore_barrier()

    @pl.loop(0, 8)
    def _copy_out(j):
        off = sid * OSHARE + j * (OSHARE // 8)
        pltpu.sync_copy(acc_sh.at[pl.ds(off, OSHARE // 8)],
                        out_hbm.at[cid].at[pl.ds(off, OSHARE // 8)])


def _edges(sidx2d, didx2d, st, dt):
    mesh = plsc.VectorSubcoreMesh(core_axis_name="c", subcore_axis_name="s")
    cp = pltpu.CompilerParams()
    if "needs_layout_passes" in pltpu.CompilerParams.__dataclass_fields__:
        cp = dataclasses.replace(cp, needs_layout_passes=False)
    if "use_tc_tiling_on_sc" in pltpu.CompilerParams.__dataclass_fields__:
        cp = dataclasses.replace(cp, use_tc_tiling_on_sc=False)
    fn = pl.kernel(
        _edge_body,
        out_type=jax.ShapeDtypeStruct((2, HALF, 16), jnp.float32),
        compiler_params=cp,
        mesh=mesh,
        scratch_types=[
            pltpu.VMEM((CROWS, 128), jnp.int32),
            pltpu.VMEM((CROWS, 128), jnp.int32),
            pltpu.VMEM((CROWS, 128), jnp.int32),
            pltpu.VMEM((CH, 8), jnp.float32),
            pltpu.VMEM((CH, 4), jnp.float32),
            pltpu.VMEM((CH, 16), jnp.float32),
            pltpu.VMEM((ZROWS, 16), jnp.float32),
            pltpu.VMEM_SHARED((ACC_ROWS, 16), jnp.float32),
            pltpu.SemaphoreType.DMA,
        ],
    )
    out = fn(sidx2d, didx2d, st, dt)
    return out.reshape(NPAD, 16)


def _final_body(ps_ref, st_ref, dt_ref, bg_ref,
                wgh_refs, w1x_refs, w1h_refs, wmid_refs, w5_refs,
                bmid_refs, b5_refs, id_ref, p_ref):
    s = ps_ref[...]
    asrc = st_ref[:, 0:4]
    xb = st_ref[:, 4:7]
    adst = dt_ref[...]
    vl = asrc + adst
    eal = jnp.exp(_lrelu(vl, NEG_GAT))
    denom = s[:, 0:4] + eal + 1e-16

    g = jnp.zeros((s.shape[0], HID), jnp.float32)
    for h in range(4):
        acc_h = s[:, 4 + 3 * h:7 + 3 * h] + eal[:, h:h + 1] * xb
        num_h = jnp.dot(acc_h, wgh_refs[h][...],
                        preferred_element_type=jnp.float32)
        g = g + num_h / denom[:, h:h + 1]
    g = g * 0.25 + bg_ref[...]
    hfeat = _lrelu(g, NEG)

    for m in range(2):
        u = jnp.dot(xb, w1x_refs[m][...], preferred_element_type=jnp.float32)
        u = u + jnp.dot(hfeat, w1h_refs[m][...],
                        preferred_element_type=jnp.float32)
        u = _lrelu(u + bmid_refs[m][:, 0:HID], NEG)
        for l in range(3):
            u = jnp.dot(u, wmid_refs[m][l][...],
                        preferred_element_type=jnp.float32)
            u = _lrelu(u + bmid_refs[m][:, (l + 1) * HID:(l + 2) * HID], NEG)
        u = jnp.dot(u, w5_refs[m][...], preferred_element_type=jnp.float32)
        u = u + b5_refs[m][...]
        if m == 0:
            id_ref[...] = u
        else:
            p_ref[...] = u


def _final_wrapped(ps_ref, st_ref, dt_ref, bg_ref,
                   wgh0, wgh1, wgh2, wgh3,
                   w1x_0, w1h_0, wm0_0, wm1_0, wm2_0, w5_0, bm_0, b5_0,
                   w1x_1, w1h_1, wm0_1, wm1_1, wm2_1, w5_1, bm_1, b5_1,
                   id_ref, p_ref):
    _final_body(ps_ref, st_ref, dt_ref, bg_ref,
                [wgh0, wgh1, wgh2, wgh3],
                [w1x_0, w1x_1], [w1h_0, w1h_1],
                [[wm0_0, wm1_0, wm2_0], [wm0_1, wm1_1, wm2_1]],
                [w5_0, w5_1], [bm_0, bm_1], [b5_0, b5_1],
                id_ref, p_ref)


def _finalize(ps, st, dt, bg, wghs, mlp1, mlp2):
    def fixed(shape):
        nd = len(shape)
        return pl.BlockSpec(shape, lambda i, _nd=nd: (0,) * _nd)

    in_specs = [
        pl.BlockSpec((BN, 16), lambda i: (i, 0)),
        pl.BlockSpec((BN, 8), lambda i: (i, 0)),
        pl.BlockSpec((BN, 4), lambda i: (i, 0)),
        fixed((1, HID)),
    ]
    args = [ps, st, dt, bg]
    for w in wghs:
        in_specs.append(fixed((3, HID)))
        args.append(w)
    for m in (mlp1, mlp2):
        w1x, w1h, wm0, wm1, wm2, w5, bm, b5 = m
        in_specs += [fixed((3, HID)), fixed((HID, HID)),
                     fixed((HID, HID)), fixed((HID, HID)), fixed((HID, HID)),
                     fixed((HID, w5.shape[1])),
                     fixed((1, 4 * HID)), fixed((1, b5.shape[1]))]
        args += [w1x, w1h, wm0, wm1, wm2, w5, bm, b5]

    return pl.pallas_call(
        _final_wrapped,
        grid=(GRID,),
        in_specs=in_specs,
        out_specs=[
            pl.BlockSpec((BN, 8), lambda i: (i, 0)),
            pl.BlockSpec((BN, 3), lambda i: (i, 0)),
        ],
        out_shape=[
            jax.ShapeDtypeStruct((NPAD, 8), jnp.float32),
            jax.ShapeDtypeStruct((NPAD, 3), jnp.float32),
        ],
    )(*args)


def kernel(x, edge_index, edge_attr, W_gat, att_src, att_dst, b_gat,
           nn1_Ws, nn1_bs, nn2_Ws, nn2_bs):
    del edge_attr

    W3 = W_gat.reshape(3, HEADS, HID)
    w_src = jnp.einsum("dhk,hk->dh", W3, att_src)
    w_dst = jnp.einsum("dhk,hk->dh", W3, att_dst)

    x_pad = jnp.zeros((NPAD, 3), jnp.float32).at[:N_NODES].set(x)
    st, dt = _prep(x_pad, w_src, w_dst)

    pad = jnp.full((EP - N_EDGES,), N_NODES, jnp.int32)
    sidx2d = jnp.concatenate([edge_index[0], pad]).reshape(EP // 128, 128)
    didx2d = jnp.concatenate([edge_index[1], pad]).reshape(EP // 128, 128)

    partials = _edges(sidx2d, didx2d, st, dt)

    def prep_mlp(Ws, bs):
        w1x = Ws[0][0:3]
        w1h = Ws[0][3:3 + HID]
        bm = jnp.concatenate([bs[0], bs[1], bs[2], bs[3]]).reshape(1, 4 * HID)
        b5 = bs[4].reshape(1, -1)
        return (w1x, w1h, Ws[1], Ws[2], Ws[3], Ws[4], bm, b5)

    wghs = [W_gat[:, h * HID:(h + 1) * HID] for h in range(HEADS)]
    id_pad, p_pad = _finalize(
        partials, st, dt, b_gat.reshape(1, HID), wghs,
        prep_mlp(nn1_Ws, nn1_bs), prep_mlp(nn2_Ws, nn2_bs))

    return (id_pad[:N_NODES], p_pad[:N_NODES])
